# baseline (speedup 1.0000x reference)
import functools

import jax
import jax.numpy as jnp
import numpy as np
from jax import lax
from jax.experimental import pallas as pl
from jax.experimental.pallas import tpu as pltpu
from jax.experimental.pallas import tpu_sc as plsc

_ALPHA = 10.0
_PIXEL = 28.0
_N_CLASSES = 11
_BG = 10
_B = 128
_NP = 1100
_NP_PAD = 1104
_ROW = _NP * (_N_CLASSES + 4)

_NW = 32
_IPW = _B // _NW
_NCHUNK = _NP_PAD // 16


def _prior_rows(n_pad):
    scales = [0.38, 0.14, 0.28, 0.11, 0.33, 0.08, 0.16, 0.12, 0.1, 0.23, 0.36]
    ratios = [0.99, 1.33, 1.96, 2.13, 1.45, 4.0, 1.004, 1.71, 2.8, 2.95, 1.21]
    pb = []
    for i in range(10):
        for j in range(10):
            cx = (j + 0.5) / 10.0
            cy = (i + 0.5) / 10.0
            for s, r in zip(scales, ratios):
                pb.append([cx, cy, s * np.sqrt(r), s / np.sqrt(r)])
    cxcy = np.clip(np.asarray(pb, dtype=np.float32), 0.0, 1.0)
    xy = np.concatenate([cxcy[:, :2] - cxcy[:, 2:] / 2.0,
                         cxcy[:, :2] + cxcy[:, 2:] / 2.0], axis=1).astype(np.float32)
    xy = np.clip(xy, 0.0, 1.0)
    area = ((xy[:, 2] - xy[:, 0]) * (xy[:, 3] - xy[:, 1])).astype(np.float32)
    rows = np.concatenate([xy.T, cxcy.T, area[None, :]], axis=0)
    out = np.zeros((9, n_pad), dtype=np.float32)
    out[:, :_NP] = rows
    out[0:4, _NP:] = 2.0
    return out


_PRIOR_ROWS_PAD = _prior_rows(_NP_PAD)


def _bfly_max(v, lane):
    for s in (8, 4, 2, 1):
        v = jnp.maximum(v, v.at[lane ^ s].get(mode="promise_in_bounds"))
    return v


def _bfly_min(v, lane):
    for s in (8, 4, 2, 1):
        v = jnp.minimum(v, v.at[lane ^ s].get(mode="promise_in_bounds"))
    return v


def _mbox_sc(pred_hbm, actual_hbm, priors_hbm, out_hbm, act_v, pr_v, img_v,
             out_v, sem):
    wid = lax.axis_index("s") * 2 + lax.axis_index("c")
    pltpu.sync_copy(actual_hbm.at[pl.ds(wid * (_IPW * 16), _IPW * 16)], act_v)
    pltpu.sync_copy(priors_hbm, pr_v)
    lane = lax.iota(jnp.int32, 16)
    out_all = jnp.zeros((16,), jnp.float32)
    for k in range(_IPW):
        i = wid * _IPW + k
        pltpu.sync_copy(pred_hbm.at[i], img_v)
        act_row = act_v[pl.ds(k * 16, 16)] / _PIXEL
        bx1 = act_row[1]
        by1 = act_row[2]
        bx2 = act_row[3]
        by2 = act_row[4]
        a1 = (bx2 - bx1) * (by2 - by1)

        def match_body(c, carry):
            m, idx = carry

            def sl_at(r):
                return pl.ds(r * _NP_PAD + c * 16, 16)
            lo_x = jnp.maximum(bx1, pr_v[sl_at(0)])
            lo_y = jnp.maximum(by1, pr_v[sl_at(1)])
            hi_x = jnp.minimum(bx2, pr_v[sl_at(2)])
            hi_y = jnp.minimum(by2, pr_v[sl_at(3)])
            inter = (jnp.maximum(hi_x - lo_x, 0.0)
                     * jnp.maximum(hi_y - lo_y, 0.0))
            union = a1 + pr_v[sl_at(8)] - inter
            iou = inter / union
            gidx = c * 16 + lane
            iou = jnp.where(gidx < _NP, iou, -1.0)
            upd = iou > m
            return jnp.where(upd, iou, m), jnp.where(upd, gidx, idx)

        m0 = jnp.full((16,), -2.0, jnp.float32)
        i0 = jnp.full((16,), _NP, jnp.int32)
        m, idx = lax.fori_loop(0, _NCHUNK, match_body, (m0, i0))
        mx = _bfly_max(m, lane)
        pfo = _bfly_min(jnp.where(m == mx, idx, _NP), lane)

        def ce_body(c, carry):
            qmax, spos = carry
            p = c * 16 + lane
            pc = jnp.minimum(p, _NP - 1)
            base = pc * (_N_CLASSES + 4)
            q = jnp.zeros((16,), jnp.float32)
            ebg = jnp.zeros((16,), jnp.float32)
            for cc in range(_N_CLASSES):
                s = plsc.load_gather(img_v, [base + cc])
                e = jnp.exp(s)
                q = q + e
                if cc == _BG:
                    ebg = e
            qq = q / ebg
            qq = jnp.where(p == pfo, 1.0, qq)
            qq = jnp.where(p < _NP, qq, 0.0)
            qmax = jnp.maximum(qmax, qq)
            spos = jnp.where(p == pfo, q, spos)
            return qmax, spos

        qmax0 = jnp.zeros((16,), jnp.float32)
        spos0 = jnp.zeros((16,), jnp.float32)
        qmax, spos = lax.fori_loop(0, _NCHUNK, ce_body, (qmax0, spos0))
        qmax = _bfly_max(qmax, lane)
        spos = _bfly_max(spos, lane)

        civ = act_v[pl.ds(k * 16, 16)].astype(jnp.int32)
        ci = civ.at[jnp.zeros((16,), jnp.int32)].get(
            mode="promise_in_bounds")
        s_ci = plsc.load_gather(img_v, [pfo * (_N_CLASSES + 4) + ci])

        gbase = pfo * (_N_CLASSES + 4)
        g0 = jnp.clip(plsc.load_gather(img_v, [gbase + 11]), 0.0, 1.0)
        g1 = jnp.clip(plsc.load_gather(img_v, [gbase + 12]), 0.0, 1.0)
        g2 = jnp.clip(plsc.load_gather(img_v, [gbase + 13]), 0.0, 1.0)
        g3 = jnp.clip(plsc.load_gather(img_v, [gbase + 14]), 0.0, 1.0)
        pcx = plsc.load_gather(pr_v, [pfo + (4 * _NP_PAD)])
        pcy = plsc.load_gather(pr_v, [pfo + (5 * _NP_PAD)])
        pw = plsc.load_gather(pr_v, [pfo + (6 * _NP_PAD)])
        ph = plsc.load_gather(pr_v, [pfo + (7 * _NP_PAD)])
        cx = g0 * pw / 10.0 + pcx
        cy = g1 * ph / 10.0 + pcy
        w = jnp.exp(g2 / 5.0) * pw
        h = jnp.exp(g3 / 5.0) * ph
        xlo = jnp.clip(cx - w / 2.0, 0.0, 1.0)
        ylo = jnp.clip(cy - h / 2.0, 0.0, 1.0)
        xhi = jnp.clip(cx + w / 2.0, 0.0, 1.0)
        yhi = jnp.clip(cy + h / 2.0, 0.0, 1.0)
        loc = (jnp.abs(xlo - bx1) + jnp.abs(ylo - by1)
               + jnp.abs(xhi - bx2) + jnp.abs(yhi - by2))

        out_all = jnp.where(lane == k, qmax, out_all)
        out_all = jnp.where(lane == _IPW + k, spos, out_all)
        out_all = jnp.where(lane == 2 * _IPW + k, s_ci, out_all)
        out_all = jnp.where(lane == 3 * _IPW + k, loc, out_all)
    out_v[...] = out_all
    pltpu.sync_copy(out_v, out_hbm.at[wid])


@functools.cache
def _get_sc_kernel():
    return functools.partial(
        pl.kernel,
        mesh=plsc.VectorSubcoreMesh(core_axis_name="c", subcore_axis_name="s"),
        compiler_params=pltpu.CompilerParams(needs_layout_passes=False),
        out_type=jax.ShapeDtypeStruct((_NW, 16), jnp.float32),
        scratch_types=[
            pltpu.VMEM((_IPW * 16,), jnp.float32),
            pltpu.VMEM((9 * _NP_PAD,), jnp.float32),
            pltpu.VMEM((_ROW,), jnp.float32),
            pltpu.VMEM((16,), jnp.float32),
            pltpu.SemaphoreType.DMA,
        ],
    )(_mbox_sc)


def _combine_tc(x_ref, out_ref):
    x = x_ref[...]
    qmax = x[:, 0:_IPW]
    spos = x[:, _IPW:2 * _IPW]
    s_ci = x[:, 2 * _IPW:3 * _IPW]
    loc = x[:, 3 * _IPW:4 * _IPW]
    conf = jnp.log(spos) - s_ci + jnp.log(qmax)
    total = (jnp.sum(conf) / _B
             + (_ALPHA / (_B * 4.0)) * jnp.sum(loc))
    out_ref[...] = total.reshape(1, 1)


@jax.jit
def kernel(pred, actual):
    pred2 = pred.reshape(_B, _ROW)
    act_pad = jnp.zeros((_B, 16), jnp.float32).at[:, :5].set(actual).reshape(_B * 16)
    priors_pad = jnp.asarray(_PRIOR_ROWS_PAD).reshape(9 * _NP_PAD)
    stats = _get_sc_kernel()(pred2, act_pad, priors_pad)
    out = pl.pallas_call(
        _combine_tc,
        out_shape=jax.ShapeDtypeStruct((1, 1), jnp.float32),
    )(stats)
    return out[0, 0]
